# baseline (speedup 1.0000x reference)
import functools

import jax
import jax.numpy as jnp
from jax import lax
from jax.experimental import pallas as pl
from jax.experimental.pallas import tpu as pltpu
from jax.experimental.pallas import tpu_sc as plsc

_BN = 256


def _argmin_body(x_ref, wT_ref, qi_ref):
    xb = x_ref[...]
    wT = wT_ref[...]
    q = wT.shape[1]
    bn = xb.shape[0]

    acc = None
    for g in range(4):
        terms = []
        for s in range(8):
            j = 8 * g + s
            d = wT[j, :][None, :] - xb[:, j][:, None]
            terms.append(d * d)
        b = [terms[s] + terms[s + 4] for s in range(4)]
        c0 = b[0] + b[2]
        c1 = b[1] + b[3]
        e = c0 + c1
        acc = e if acc is None else acc + e

    m = jnp.min(acc, axis=1, keepdims=True)
    iota = jax.lax.broadcasted_iota(jnp.int32, (bn, q), 1)
    idx = jnp.min(jnp.where(acc == m, iota, q), axis=1)
    qi_ref[...] = idx[:, None]


def _make_sc_gather(v, d, b):
    info = plsc.get_sparse_core_info()
    nw = info.num_cores * info.num_subcores
    nc = info.num_cores
    b_per_w = b // nw
    mesh = plsc.VectorSubcoreMesh(core_axis_name="c", subcore_axis_name="s")

    @functools.partial(
        pl.kernel, mesh=mesh,
        compiler_params=pltpu.CompilerParams(use_tc_tiling_on_sc=False),
        out_type=jax.ShapeDtypeStruct((b, d), jnp.float32),
        scratch_types=[
            pltpu.VMEM((b_per_w,), jnp.int32),
            pltpu.VMEM((b_per_w, d), jnp.float32),
            pltpu.SemaphoreType.DMA,
        ],
    )
    def sc_gather(table_hbm, idx_hbm, out_hbm, idx_v, rows_v, sem):
        wid = lax.axis_index("s") * nc + lax.axis_index("c")
        base = wid * b_per_w
        pltpu.sync_copy(idx_hbm.at[pl.ds(base, b_per_w)], idx_v)
        pltpu.async_copy(table_hbm.at[idx_v], rows_v, sem).wait()
        pltpu.sync_copy(rows_v, out_hbm.at[pl.ds(base, b_per_w)])

    return sc_gather


def kernel(x, weight):
    input_shape = x.shape
    e = weight.shape[1]
    q = weight.shape[0]
    xf = x.reshape(-1, e)
    n = xf.shape[0]
    wT = weight.T

    q_idx = pl.pallas_call(
        _argmin_body,
        grid=(n // _BN,),
        in_specs=[
            pl.BlockSpec((_BN, e), lambda i: (i, 0)),
            pl.BlockSpec((e, q), lambda i: (0, 0)),
        ],
        out_specs=pl.BlockSpec((_BN, 1), lambda i: (i, 0)),
        out_shape=jax.ShapeDtypeStruct((n, 1), jnp.int32),
    )(xf, wT)
    idx_flat = q_idx.reshape(n)

    q_data = _make_sc_gather(q, e, n)(weight, idx_flat)
    return (q_data.reshape(input_shape),
            idx_flat.reshape(input_shape[:-1]))


# speedup vs baseline: 1.1262x; 1.1262x over previous
import jax
import jax.numpy as jnp
from jax.experimental import pallas as pl
from jax.experimental.pallas import tpu as pltpu

_BN = 256


def _vq_body(x_ref, w_ref, qd_ref, qi_ref,
             wT_ref, whi_ref, wmid_ref, wlo_ref):
    @pl.when(pl.program_id(0) == 0)
    def _setup():
        w = w_ref[...]
        wT_ref[...] = w.T
        hi = w.astype(jnp.bfloat16)
        r1 = w - hi.astype(jnp.float32)
        r1s = r1 * 256.0
        mid = r1s.astype(jnp.bfloat16)
        r2s = (r1s - mid.astype(jnp.float32)) * 256.0
        whi_ref[...] = hi
        wmid_ref[...] = mid
        wlo_ref[...] = r2s.astype(jnp.bfloat16)

    xb = x_ref[...]
    wT = wT_ref[...]
    q = wT.shape[1]
    bn = xb.shape[0]

    acc = None
    for g in range(4):
        terms = []
        for s in range(8):
            j = 8 * g + s
            d = wT[j, :][None, :] - xb[:, j][:, None]
            terms.append(d * d)
        b = [terms[s] + terms[s + 4] for s in range(4)]
        c0 = b[0] + b[2]
        c1 = b[1] + b[3]
        e = c0 + c1
        acc = e if acc is None else acc + e

    m = jnp.min(acc, axis=1, keepdims=True)
    iota = jax.lax.broadcasted_iota(jnp.int32, (bn, q), 1)
    idx = jnp.min(jnp.where(acc == m, iota, q), axis=1)
    seg = pl.program_id(0) % (qi_ref.shape[2] // bn)
    qi_ref[0, 0, pl.ds(seg * bn, bn)] = idx

    onehot = (iota == idx[:, None]).astype(jnp.bfloat16)
    dims = (((1,), (0,)), ((), ()))
    g_hi = jax.lax.dot_general(onehot, whi_ref[...], dims,
                               preferred_element_type=jnp.float32)
    g_mid = jax.lax.dot_general(onehot, wmid_ref[...], dims,
                                preferred_element_type=jnp.float32)
    g_lo = jax.lax.dot_general(onehot, wlo_ref[...], dims,
                               preferred_element_type=jnp.float32)
    qd_ref[...] = g_hi + (g_mid * (1.0 / 256.0) + g_lo * (1.0 / 65536.0))


def kernel(x, weight):
    input_shape = x.shape
    e = weight.shape[1]
    q = weight.shape[0]
    xf = x.reshape(-1, e)
    n = xf.shape[0]
    rows = input_shape[0]
    per_row = n // rows // _BN

    q_data, q_idx = pl.pallas_call(
        _vq_body,
        grid=(n // _BN,),
        in_specs=[
            pl.BlockSpec((_BN, e), lambda i: (i, 0)),
            pl.BlockSpec((q, e), lambda i: (0, 0)),
        ],
        out_specs=[
            pl.BlockSpec((_BN, e), lambda i: (i, 0)),
            pl.BlockSpec((1, 1, n // rows), lambda i: (i // per_row, 0, 0)),
        ],
        out_shape=[
            jax.ShapeDtypeStruct((n, e), jnp.float32),
            jax.ShapeDtypeStruct((rows, 1, n // rows), jnp.int32),
        ],
        scratch_shapes=[
            pltpu.VMEM((e, q), jnp.float32),
            pltpu.VMEM((q, e), jnp.bfloat16),
            pltpu.VMEM((q, e), jnp.bfloat16),
            pltpu.VMEM((q, e), jnp.bfloat16),
        ],
    )(xf, weight)
    return (q_data.reshape(input_shape), q_idx.reshape(input_shape[:-1]))


# speedup vs baseline: 1.1528x; 1.0236x over previous
import jax
import jax.numpy as jnp
from jax.experimental import pallas as pl
from jax.experimental.pallas import tpu as pltpu

_BN = 512


def _vq_body(x_ref, w_ref, qd_ref, qi_ref,
             wT_ref, whi_ref, wmid_ref, wlo_ref):
    @pl.when(pl.program_id(0) == 0)
    def _setup():
        w = w_ref[...]
        wT_ref[...] = w.T
        hi = w.astype(jnp.bfloat16)
        r1 = w - hi.astype(jnp.float32)
        r1s = r1 * 256.0
        mid = r1s.astype(jnp.bfloat16)
        r2s = (r1s - mid.astype(jnp.float32)) * 256.0
        whi_ref[...] = hi
        wmid_ref[...] = mid
        wlo_ref[...] = r2s.astype(jnp.bfloat16)

    xb = x_ref[...]
    wT = wT_ref[...]
    q = wT.shape[1]
    bn = xb.shape[0]

    acc = None
    for g in range(4):
        terms = []
        for s in range(8):
            j = 8 * g + s
            d = wT[j, :][None, :] - xb[:, j][:, None]
            terms.append(d * d)
        b = [terms[s] + terms[s + 4] for s in range(4)]
        c0 = b[0] + b[2]
        c1 = b[1] + b[3]
        e = c0 + c1
        acc = e if acc is None else acc + e

    m = jnp.min(acc, axis=1, keepdims=True)
    iota = jax.lax.broadcasted_iota(jnp.int32, (bn, q), 1)
    idx = jnp.min(jnp.where(acc == m, iota, q), axis=1)
    seg = pl.program_id(0) % (qi_ref.shape[2] // bn)
    qi_ref[0, 0, pl.ds(seg * bn, bn)] = idx

    onehot = (iota == idx[:, None]).astype(jnp.bfloat16)
    dims = (((1,), (0,)), ((), ()))
    g_hi = jax.lax.dot_general(onehot, whi_ref[...], dims,
                               preferred_element_type=jnp.float32)
    g_mid = jax.lax.dot_general(onehot, wmid_ref[...], dims,
                                preferred_element_type=jnp.float32)
    g_lo = jax.lax.dot_general(onehot, wlo_ref[...], dims,
                               preferred_element_type=jnp.float32)
    qd_ref[...] = g_hi + (g_mid * (1.0 / 256.0) + g_lo * (1.0 / 65536.0))


def kernel(x, weight):
    input_shape = x.shape
    e = weight.shape[1]
    q = weight.shape[0]
    xf = x.reshape(-1, e)
    n = xf.shape[0]
    rows = input_shape[0]
    per_row = n // rows // _BN

    q_data, q_idx = pl.pallas_call(
        _vq_body,
        grid=(n // _BN,),
        in_specs=[
            pl.BlockSpec((_BN, e), lambda i: (i, 0)),
            pl.BlockSpec((q, e), lambda i: (0, 0)),
        ],
        out_specs=[
            pl.BlockSpec((_BN, e), lambda i: (i, 0)),
            pl.BlockSpec((1, 1, n // rows), lambda i: (i // per_row, 0, 0)),
        ],
        out_shape=[
            jax.ShapeDtypeStruct((n, e), jnp.float32),
            jax.ShapeDtypeStruct((rows, 1, n // rows), jnp.int32),
        ],
        scratch_shapes=[
            pltpu.VMEM((e, q), jnp.float32),
            pltpu.VMEM((q, e), jnp.bfloat16),
            pltpu.VMEM((q, e), jnp.bfloat16),
            pltpu.VMEM((q, e), jnp.bfloat16),
        ],
    )(xf, weight)
    return (q_data.reshape(input_shape), q_idx.reshape(input_shape[:-1]))
